# baseline (speedup 1.0000x reference)
import functools

import jax
import jax.numpy as jnp
from jax import lax
from jax.experimental import pallas as pl
from jax.experimental.pallas import tpu as pltpu
from jax.experimental.pallas import tpu_sc as plsc

_INFO = plsc.get_sparse_core_info()
_NC = _INFO.num_cores
_NS = _INFO.num_subcores
_NW = _NC * _NS


@functools.cache
def _make_gather(n_rows: int, d: int, chunk: int):
    assert n_rows % (_NW * chunk) == 0 and chunk % 8 == 0
    per_w = n_rows // _NW
    nchunk = per_w // chunk
    mesh = plsc.VectorSubcoreMesh(core_axis_name="c", subcore_axis_name="s")

    @functools.partial(
        pl.kernel,
        mesh=mesh,
        out_type=jax.ShapeDtypeStruct((n_rows, d), jnp.float32),
        compiler_params=pltpu.CompilerParams(use_tc_tiling_on_sc=False),
        scratch_types=(
            [pltpu.VMEM((chunk,), jnp.int32) for _ in range(nchunk)]
            + [
                pltpu.VMEM((chunk, d), jnp.float32),
                pltpu.VMEM((chunk, d), jnp.float32),
                pltpu.SemaphoreType.DMA,
                pltpu.SemaphoreType.DMA,
                pltpu.SemaphoreType.DMA,
                pltpu.SemaphoreType.DMA,
            ]
        ),
    )
    def gather(table_hbm, idx_hbm, out_hbm, *scratch):
        idx_vs = scratch[:nchunk]
        buf0, buf1, g0, g1, o0, o1 = scratch[nchunk:]
        wid = lax.axis_index("s") * _NC + lax.axis_index("c")
        base = wid * per_w
        for i in range(nchunk):
            pltpu.sync_copy(idx_hbm.at[wid, i], idx_vs[i])
        bufs = (buf0, buf1)
        gsems = (g0, g1)
        osems = (o0, o1)
        gathers = [None] * nchunk
        outs = [None] * nchunk
        gathers[0] = pltpu.async_copy(table_hbm.at[idx_vs[0]], bufs[0],
                                      gsems[0])
        for i in range(nchunk):
            cur = i % 2
            nxt = (i + 1) % 2
            if i + 1 < nchunk:
                if i >= 1:
                    outs[i - 1].wait()
                gathers[i + 1] = pltpu.async_copy(
                    table_hbm.at[idx_vs[i + 1]], bufs[nxt], gsems[nxt])
            gathers[i].wait()
            outs[i] = pltpu.async_copy(
                bufs[cur], out_hbm.at[pl.ds(base + i * chunk, chunk)],
                osems[cur])
        if nchunk >= 2:
            outs[nchunk - 2].wait()
        outs[nchunk - 1].wait()

    return gather


def kernel(token_ids, std_embed, q6_basis, hamming_scale):
    b, t = token_ids.shape
    d = std_embed.shape[1]
    n_rows = b * t
    chunk = 640
    per_w = n_rows // _NW
    idx = token_ids.reshape(_NW, per_w // chunk, chunk).astype(jnp.int32)
    out = _make_gather(n_rows, d, chunk)(std_embed, idx)
    return (out.reshape(b, t, d), None)


# speedup vs baseline: 1.2322x; 1.2322x over previous
import functools

import jax
import jax.numpy as jnp
from jax import lax
from jax.experimental import pallas as pl
from jax.experimental.pallas import tpu as pltpu
from jax.experimental.pallas import tpu_sc as plsc

_INFO = plsc.get_sparse_core_info()
_NC = _INFO.num_cores
_NS = _INFO.num_subcores
_NW = _NC * _NS


@functools.cache
def _make_transpose(v_pad, d, bc):
    nblk = v_pad // (2 * bc)

    def body(a_ref, out_ref):
        x = a_ref[...]
        out_ref[...] = jnp.concatenate(
            [x[:, :bc].T, x[:, bc:].T], axis=1)

    return pl.pallas_call(
        body,
        grid=(nblk,),
        in_specs=[pl.BlockSpec((d, 2 * bc), lambda i: (0, i))],
        out_specs=pl.BlockSpec((bc, 2 * d), lambda i: (i, 0)),
        out_shape=jax.ShapeDtypeStruct((v_pad // 2, 2 * d), jnp.float32),
    )


@functools.cache
def _make_gather(n_rows: int, d: int, chunk: int):
    per_w = n_rows // _NW
    nchunk = per_w // chunk
    mesh = plsc.VectorSubcoreMesh(core_axis_name="c", subcore_axis_name="s")

    @functools.partial(
        pl.kernel,
        mesh=mesh,
        out_type=jax.ShapeDtypeStruct((n_rows, d), jnp.float32),
        compiler_params=pltpu.CompilerParams(use_tc_tiling_on_sc=False),
        scratch_types=(
            [pltpu.VMEM((chunk,), jnp.int32) for _ in range(nchunk)]
            + [
                pltpu.VMEM((chunk, d), jnp.float32),
                pltpu.VMEM((chunk, d), jnp.float32),
                pltpu.SemaphoreType.DMA,
                pltpu.SemaphoreType.DMA,
                pltpu.SemaphoreType.DMA,
                pltpu.SemaphoreType.DMA,
            ]
        ),
    )
    def gather(table_hbm, idx_hbm, out_hbm, *scratch):
        idx_vs = scratch[:nchunk]
        buf0, buf1, g0, g1, o0, o1 = scratch[nchunk:]
        wid = lax.axis_index("s") * _NC + lax.axis_index("c")
        base = wid * per_w
        for i in range(nchunk):
            pltpu.sync_copy(idx_hbm.at[wid, i], idx_vs[i])
        bufs = (buf0, buf1)
        gsems = (g0, g1)
        osems = (o0, o1)
        gathers = [None] * nchunk
        outs = [None] * nchunk
        gathers[0] = pltpu.async_copy(table_hbm.at[idx_vs[0]], bufs[0],
                                      gsems[0])
        for i in range(nchunk):
            cur = i % 2
            nxt = (i + 1) % 2
            if i + 1 < nchunk:
                if i >= 1:
                    outs[i - 1].wait()
                gathers[i + 1] = pltpu.async_copy(
                    table_hbm.at[idx_vs[i + 1]], bufs[nxt], gsems[nxt])
            gathers[i].wait()
            outs[i] = pltpu.async_copy(
                bufs[cur], out_hbm.at[pl.ds(base + i * chunk, chunk)],
                osems[cur])
        if nchunk >= 2:
            outs[nchunk - 2].wait()
        outs[nchunk - 1].wait()

    return gather


def kernel(token_ids, std_embed, q6_basis, hamming_scale):
    b, t = token_ids.shape
    v, d = std_embed.shape
    n_rows = b * t
    chunk = 640
    per_w = n_rows // _NW
    bc = 1024
    v_pad = -(-v // (2 * bc)) * (2 * bc)
    table128 = _make_transpose(v_pad, d, bc)(std_embed.T)
    table = table128.reshape(v_pad, d)
    flat = token_ids.reshape(-1).astype(jnp.int32)
    blk = flat >> 11
    j = flat & (2 * bc - 1)
    remapped = ((blk * bc + (j & (bc - 1))) << 1) | (j >> 10)
    idx = remapped.reshape(_NW, per_w // chunk, chunk)
    out = _make_gather(n_rows, d, chunk)(table, idx)
    return (out.reshape(b, t, d), None)
